# Initial kernel scaffold; baseline (speedup 1.0000x reference)
import jax
import jax.numpy as jnp
from jax.experimental import pallas as pl


def kernel(template, projections):
    raise NotImplementedError("write your pallas kernel here")


# speedup vs baseline: 87.1729x; 87.1729x over previous
import functools
from itertools import combinations

import jax
import jax.numpy as jnp
import numpy as np
from jax.experimental import pallas as pl

N_RADIAL, N_ANGULAR = 5, 8
N_VERTICES, N_NEIGHBORS = 2048, 8
N_PTS = N_RADIAL * N_ANGULAR
TRI = np.array(list(combinations(range(N_NEIGHBORS), 3)), dtype=np.int32)
N_TRI = TRI.shape[0]
T_PAD = 64
BLK_V = 128

_INF = np.float32(np.inf)


def _onehot(col):
    m = np.zeros((N_NEIGHBORS, T_PAD), dtype=np.float32)
    m[TRI[:, col], np.arange(N_TRI)] = 1.0
    return m


_OH_A, _OH_B, _OH_C = _onehot(0), _onehot(1), _onehot(2)
_I0 = np.zeros((T_PAD,), np.int32)
_I1 = np.zeros((T_PAD,), np.int32)
_I2 = np.zeros((T_PAD,), np.int32)
_I0[:N_TRI], _I1[:N_TRI], _I2[:N_TRI] = TRI[:, 0], TRI[:, 1], TRI[:, 2]


def _body(px_ref, py_ref, tx_ref, ty_ref, oh_ref, itab_ref,
          w0_ref, w1_ref, w2_ref, i0_ref, i1_ref, i2_ref):
    px = px_ref[...]
    py = py_ref[...]
    tx = tx_ref[...][0]
    ty = ty_ref[...][0]

    oh_a = oh_ref[0]
    oh_b = oh_ref[1]
    oh_c = oh_ref[2]
    hp = jax.lax.Precision.HIGHEST
    ax = jnp.dot(px, oh_a, precision=hp); ay = jnp.dot(py, oh_a, precision=hp)
    bx = jnp.dot(px, oh_b, precision=hp); by = jnp.dot(py, oh_b, precision=hp)
    cx = jnp.dot(px, oh_c, precision=hp); cy = jnp.dot(py, oh_c, precision=hp)

    v0x = cx - ax; v0y = cy - ay
    v1x = bx - ax; v1y = by - ay
    orient = v1x * v0y - v1y * v0x
    d00 = v0x * v0x + v0y * v0y
    d01 = v0x * v1x + v0y * v1y
    d11 = v1x * v1x + v1y * v1y
    invden = 1.0 / (d00 * d11 - d01 * d01)

    delaunay = jnp.zeros(ax.shape, dtype=jnp.bool_)
    for p in range(N_NEIGHBORS):
        pxp = px[:, p:p + 1]; pyp = py[:, p:p + 1]
        dxa = ax - pxp; dya = ay - pyp
        dxb = bx - pxp; dyb = by - pyp
        dxc = cx - pxp; dyc = cy - pyp
        za = dxa * dxa + dya * dya
        zb = dxb * dxb + dyb * dyb
        zc = dxc * dxc + dyc * dyc
        det = (dxa * (dyb * zc - dyc * zb)
               - dya * (dxb * zc - dxc * zb)
               + za * (dxb * dyc - dxc * dyb))
        delaunay = delaunay | (orient * det > 0.0)
    pen = jnp.where(delaunay, _INF, jnp.float32(0.0))

    txb = tx[None, :, None]; tyb = ty[None, :, None]
    v2x = txb - ax[:, None, :]; v2y = tyb - ay[:, None, :]
    dot02 = v0x[:, None, :] * v2x + v0y[:, None, :] * v2y
    dot12 = v1x[:, None, :] * v2x + v1y[:, None, :] * v2y
    w2 = (d11[:, None, :] * dot02 - d01[:, None, :] * dot12) * invden[:, None, :]
    w1 = (d00[:, None, :] * dot12 - d01[:, None, :] * dot02) * invden[:, None, :]
    w0 = 1.0 - w1 - w2
    wmin = jnp.minimum(jnp.minimum(w0, w1), w2)
    wmax = jnp.maximum(jnp.maximum(w0, w1), w2)
    inside = (wmin > 0.0) & (wmax < 1.0)

    da = jnp.sqrt(v2x * v2x + v2y * v2y)
    ubx = txb - bx[:, None, :]; uby = tyb - by[:, None, :]
    db = jnp.sqrt(ubx * ubx + uby * uby)
    ucx = txb - cx[:, None, :]; ucy = tyb - cy[:, None, :]
    dc = jnp.sqrt(ucx * ucx + ucy * ucy)
    dist = da + db + dc + pen[:, None, :]

    lane = jax.lax.broadcasted_iota(jnp.int32, dist.shape, 2)
    valid = inside & (lane < N_TRI)
    dist = jnp.where(valid, dist, _INF)

    mind = jnp.min(dist, axis=-1)
    eq = dist == mind[..., None]
    closest = jnp.min(jnp.where(eq, lane, T_PAD), axis=-1)
    onehot = lane == closest[..., None]

    allneg = jnp.isinf(mind)
    zf = jnp.float32(0.0)

    def selw(w):
        s = jnp.sum(jnp.where(onehot, w, zf), axis=-1)
        return jnp.where(allneg, zf, s)

    w0_ref[...] = selw(w0)
    w1_ref[...] = selw(w1)
    w2_ref[...] = selw(w2)

    zi = jnp.int32(0)

    def seli(iv):
        ivb = iv[None, None, :]
        s = jnp.sum(jnp.where(onehot, ivb, zi), axis=-1, dtype=jnp.int32)
        return jnp.where(allneg, zi, s)

    i0_ref[...] = seli(itab_ref[0])
    i1_ref[...] = seli(itab_ref[1])
    i2_ref[...] = seli(itab_ref[2])


@jax.jit
def _run(template, projections):
    px = projections[..., 0]
    py = projections[..., 1]
    t2 = template.reshape(N_PTS, 2)
    tx = t2[:, 0][None, :]
    ty = t2[:, 1][None, :]

    oh = jnp.asarray(np.stack([_OH_A, _OH_B, _OH_C]))
    itab = jnp.asarray(np.stack([_I0, _I1, _I2]))

    grid = (N_VERTICES // BLK_V,)
    bs_p = pl.BlockSpec((BLK_V, N_NEIGHBORS), lambda i: (i, i * 0))
    bs_t = pl.BlockSpec((1, N_PTS), lambda i: (i * 0, i * 0))
    bs_oh = pl.BlockSpec((3, N_NEIGHBORS, T_PAD), lambda i: (i * 0, i * 0, i * 0))
    bs_it = pl.BlockSpec((3, T_PAD), lambda i: (i * 0, i * 0))
    bs_o = pl.BlockSpec((BLK_V, N_PTS), lambda i: (i, i * 0))
    outs = pl.pallas_call(
        _body,
        grid=grid,
        in_specs=[bs_p, bs_p, bs_t, bs_t, bs_oh, bs_it],
        out_specs=[bs_o] * 6,
        out_shape=[jax.ShapeDtypeStruct((N_VERTICES, N_PTS), jnp.float32)] * 3
        + [jax.ShapeDtypeStruct((N_VERTICES, N_PTS), jnp.int32)] * 3,
    )(px, py, tx, ty, oh, itab)
    w0, w1, w2, i0, i1, i2 = outs
    bc = jnp.stack([w0, w1, w2], axis=-1)
    bc = bc.reshape(N_VERTICES, N_RADIAL, N_ANGULAR, 3).astype(jnp.float64)
    idx = jnp.stack([i0, i1, i2], axis=-1)
    idx = idx.reshape(N_VERTICES, N_RADIAL, N_ANGULAR, 3)
    return bc, idx


def kernel(template, projections):
    return _run(template, projections)
